# baseline (speedup 1.0000x reference)
import functools
import jax
import jax.numpy as jnp
from jax import lax
from jax.experimental import pallas as pl
from jax.experimental.pallas import tpu as pltpu, tpu_sc as plsc

_N_FIELDS = 26
_VOCAB = 100000
_EMB_DIM = 16
_BATCH = 16384

_NC = 2
_NS = 16
_NW = _NC * _NS

_B_PER_W = _BATCH // _NW
_CHUNK_B = 64
_N_CHUNKS = _B_PER_W // _CHUNK_B
_ROWS_PER_CHUNK = _CHUNK_B * _N_FIELDS
_IDX_MINOR = 128
_GATHERS_PER_CHUNK = _ROWS_PER_CHUNK // _IDX_MINOR
_IDX_ROWS_PER_W = _B_PER_W * _N_FIELDS // _IDX_MINOR


def _sc_body(tbl_hbm, idx_hbm, w_hbm, out_hbm, idx_v, rows_v, w_v, out_v, t_v,
             sem):
    wid = lax.axis_index("s") * _NC + lax.axis_index("c")

    pltpu.sync_copy(idx_hbm.at[pl.ds(wid * _IDX_ROWS_PER_W, _IDX_ROWS_PER_W)],
                    idx_v)
    pltpu.sync_copy(w_hbm, w_v)
    w_regs = [w_v[f] for f in range(_N_FIELDS)]

    for c in range(_N_CHUNKS):
        copies = []
        for j in range(_GATHERS_PER_CHUNK):
            copies.append(pltpu.async_copy(
                tbl_hbm.at[idx_v.at[c * _GATHERS_PER_CHUNK + j]],
                rows_v.at[pl.ds(j * _IDX_MINOR, _IDX_MINOR)],
                sem))
        for cp in copies:
            cp.wait()

        lane = lax.iota(jnp.int32, 16)

        def group(g, carry):
            def body(k, carry2):
                base = (g * 16 + k) * _N_FIELDS
                acc = rows_v[base] * w_regs[0]
                for f in range(1, _N_FIELDS):
                    acc = acc + rows_v[base + f] * w_regs[f]
                plsc.store_scatter(t_v, [lane, jnp.full((16,), 0, jnp.int32) + k], acc)
                return carry2

            lax.fori_loop(0, 16, body, 0)
            r = t_v[0]
            for i in range(1, 16):
                r = r + t_v[i]
            out_v[pl.ds(c * _CHUNK_B + g * 16, 16)] = r
            return carry

        lax.fori_loop(0, _CHUNK_B // 16, group, 0)

    pltpu.sync_copy(out_v, out_hbm.at[pl.ds(wid * _B_PER_W, _B_PER_W)])


@jax.jit
def _run(tbl, idx, wmat):
    mesh = plsc.VectorSubcoreMesh(core_axis_name="c", subcore_axis_name="s")
    f = pl.kernel(
        _sc_body,
        mesh=mesh,
        compiler_params=pltpu.CompilerParams(
            needs_layout_passes=False, use_tc_tiling_on_sc=False),
        out_type=jax.ShapeDtypeStruct((_BATCH,), jnp.float32),
        scratch_types=[
            pltpu.VMEM((_IDX_ROWS_PER_W, _IDX_MINOR), jnp.int32),
            pltpu.VMEM((_ROWS_PER_CHUNK, _EMB_DIM), jnp.float32),
            pltpu.VMEM((_N_FIELDS, _EMB_DIM), jnp.float32),
            pltpu.VMEM((_B_PER_W,), jnp.float32),
            pltpu.VMEM((16, 16), jnp.float32),
            pltpu.SemaphoreType.DMA,
        ],
    )
    return f(tbl, idx, wmat)


def kernel(X, tables, alpha):
    w = jnp.tanh(alpha).astype(jnp.float32)
    wmat = jnp.broadcast_to(w[:, None], (_N_FIELDS, _EMB_DIM))
    offs = (jnp.arange(_N_FIELDS, dtype=jnp.int32) * _VOCAB)[None, :]
    flat_idx = (X + offs).reshape(_BATCH * _N_FIELDS // _IDX_MINOR, _IDX_MINOR)
    tbl = tables.reshape(_N_FIELDS * _VOCAB, _EMB_DIM)
    out = _run(tbl, flat_idx, wmat)
    return out[:, None]


# speedup vs baseline: 4.5086x; 4.5086x over previous
import jax
import jax.numpy as jnp
from jax import lax
from jax.experimental import pallas as pl
from jax.experimental.pallas import tpu as pltpu, tpu_sc as plsc

_N_FIELDS = 26
_VOCAB = 100000
_EMB_DIM = 16
_BATCH = 16384

_NC = 2
_NS = 16
_NW = _NC * _NS

_B_PER_W = _BATCH // _NW
_IDX_MINOR = 128
_IDX_ROWS_PER_W = _B_PER_W * _N_FIELDS // _IDX_MINOR
_GATHER_BATCH = 13
_G_WORDS = _B_PER_W * _N_FIELDS
_G_PAD = _G_WORDS + 16


def _tc_reduce_body(t_ref, o_ref):
    o_ref[...] = jnp.sum(t_ref[...], axis=1, keepdims=True)


def _sc_body(s_hbm, idx_hbm, w_hbm, out_hbm, idx_v, g_v, w_v, out_v, t_v, sem):
    wid = lax.axis_index("s") * _NC + lax.axis_index("c")

    pltpu.sync_copy(idx_hbm.at[pl.ds(wid * _IDX_ROWS_PER_W, _IDX_ROWS_PER_W)],
                    idx_v)
    pltpu.sync_copy(w_hbm, w_v)
    w_a = w_v[0]
    w_b = w_v[1]
    g_v[pl.ds(_G_WORDS, 16)] = jnp.zeros((16,), jnp.float32)

    for batch in range(_IDX_ROWS_PER_W // _GATHER_BATCH):
        copies = []
        for r in range(_GATHER_BATCH):
            row = batch * _GATHER_BATCH + r
            copies.append(pltpu.async_copy(
                s_hbm.at[idx_v.at[row]],
                g_v.at[pl.ds(row * _IDX_MINOR, _IDX_MINOR)],
                sem))
        for cp in copies:
            cp.wait()

    lane = lax.iota(jnp.int32, 16)

    def group(g, carry):
        def body(k, carry2):
            base = (g * 16 + k) * _N_FIELDS
            v1 = plsc.load_gather(g_v, [base + lane])
            v2 = plsc.load_gather(g_v, [base + 16 + lane])
            v = v1 * w_a + v2 * w_b
            plsc.store_scatter(t_v, [lane, jnp.full((16,), 0, jnp.int32) + k], v)
            return carry2

        lax.fori_loop(0, 16, body, 0)
        r = t_v[0]
        for i in range(1, 16):
            r = r + t_v[i]
        out_v[pl.ds(g * 16, 16)] = r
        return carry

    lax.fori_loop(0, _B_PER_W // 16, group, 0)

    pltpu.sync_copy(out_v, out_hbm.at[pl.ds(wid * _B_PER_W, _B_PER_W)])


@jax.jit
def _run(tbl_t, idx, wmat):
    s = pl.pallas_call(
        _tc_reduce_body,
        grid=(_N_FIELDS,),
        in_specs=[pl.BlockSpec((1, _EMB_DIM, _VOCAB), lambda f: (f, 0, 0))],
        out_specs=pl.BlockSpec((1, 1, _VOCAB), lambda f: (f, 0, 0)),
        out_shape=jax.ShapeDtypeStruct((_N_FIELDS, 1, _VOCAB), jnp.float32),
    )(tbl_t)
    s_flat = s.reshape(_N_FIELDS * _VOCAB)

    mesh = plsc.VectorSubcoreMesh(core_axis_name="c", subcore_axis_name="s")
    f = pl.kernel(
        _sc_body,
        mesh=mesh,
        compiler_params=pltpu.CompilerParams(
            needs_layout_passes=False, use_tc_tiling_on_sc=False),
        out_type=jax.ShapeDtypeStruct((_BATCH,), jnp.float32),
        scratch_types=[
            pltpu.VMEM((_IDX_ROWS_PER_W, _IDX_MINOR), jnp.int32),
            pltpu.VMEM((_G_PAD,), jnp.float32),
            pltpu.VMEM((2, 16), jnp.float32),
            pltpu.VMEM((_B_PER_W,), jnp.float32),
            pltpu.VMEM((16, 16), jnp.float32),
            pltpu.SemaphoreType.DMA,
        ],
    )
    return f(s_flat, idx, wmat)


def kernel(X, tables, alpha):
    w = jnp.tanh(alpha).astype(jnp.float32)
    wmat = jnp.concatenate([w, jnp.zeros((6,), jnp.float32)]).reshape(2, 16)
    tbl_t = jnp.transpose(tables, (0, 2, 1))
    offs = (jnp.arange(_N_FIELDS, dtype=jnp.int32) * _VOCAB)[None, :]
    flat_idx = (X + offs).reshape(_BATCH * _N_FIELDS // _IDX_MINOR, _IDX_MINOR)
    out = _run(tbl_t, flat_idx, wmat)
    return out[:, None]


# speedup vs baseline: 8.4176x; 1.8670x over previous
import jax
import jax.numpy as jnp
from jax import lax
from jax.experimental import pallas as pl
from jax.experimental.pallas import tpu as pltpu, tpu_sc as plsc

_N_FIELDS = 26
_VOCAB = 100000
_EMB_DIM = 16
_BATCH = 16384

_NC = 2
_NS = 16
_NW = _NC * _NS

_B_PER_W = _BATCH // _NW
_IDX_MINOR = 128
_IDX_ROWS_PER_W = _B_PER_W * _N_FIELDS // _IDX_MINOR
_GATHER_BATCH = 13
_G_WORDS = _B_PER_W * _N_FIELDS
_G_PAD = _G_WORDS + 16


_V_CHUNK = 4096


def _tc_reduce_body(t_ref, o_ref):
    o_ref[...] = jnp.sum(t_ref[...], axis=1)


def _sc_body(s_hbm, idx_hbm, w_hbm, out_hbm, idx_v, g_v, w_v, out_v, t_v, sem):
    wid = lax.axis_index("s") * _NC + lax.axis_index("c")

    pltpu.sync_copy(idx_hbm.at[pl.ds(wid * _IDX_ROWS_PER_W, _IDX_ROWS_PER_W)],
                    idx_v)
    pltpu.sync_copy(w_hbm, w_v)
    w_a = w_v[0]
    w_b = w_v[1]
    g_v[pl.ds(_G_WORDS, 16)] = jnp.zeros((16,), jnp.float32)

    for batch in range(_IDX_ROWS_PER_W // _GATHER_BATCH):
        copies = []
        for r in range(_GATHER_BATCH):
            row = batch * _GATHER_BATCH + r
            copies.append(pltpu.async_copy(
                s_hbm.at[idx_v.at[row]],
                g_v.at[pl.ds(row * _IDX_MINOR, _IDX_MINOR)],
                sem))
        for cp in copies:
            cp.wait()

    lane = lax.iota(jnp.int32, 16)

    def group(g, carry):
        def body(k, carry2):
            base = (g * 16 + k) * _N_FIELDS
            v1 = plsc.load_gather(g_v, [base + lane])
            v2 = plsc.load_gather(g_v, [base + 16 + lane])
            v = v1 * w_a + v2 * w_b
            plsc.store_scatter(t_v, [lane, jnp.full((16,), 0, jnp.int32) + k], v)
            return carry2

        lax.fori_loop(0, 16, body, 0)
        r = t_v[0]
        for i in range(1, 16):
            r = r + t_v[i]
        out_v[pl.ds(g * 16, 16)] = r
        return carry

    lax.fori_loop(0, _B_PER_W // 16, group, 0)

    pltpu.sync_copy(out_v, out_hbm.at[pl.ds(wid * _B_PER_W, _B_PER_W)])


@jax.jit
def _run(tbl_t, idx, wmat):
    s = pl.pallas_call(
        _tc_reduce_body,
        grid=(pl.cdiv(_VOCAB, _V_CHUNK),),
        in_specs=[pl.BlockSpec((_N_FIELDS, _EMB_DIM, _V_CHUNK),
                               lambda c: (0, 0, c))],
        out_specs=pl.BlockSpec((_N_FIELDS, _V_CHUNK), lambda c: (0, c)),
        out_shape=jax.ShapeDtypeStruct((_N_FIELDS, _VOCAB), jnp.float32),
    )(tbl_t)
    s_flat = s.reshape(_N_FIELDS * _VOCAB)

    mesh = plsc.VectorSubcoreMesh(core_axis_name="c", subcore_axis_name="s")
    f = pl.kernel(
        _sc_body,
        mesh=mesh,
        compiler_params=pltpu.CompilerParams(
            needs_layout_passes=False, use_tc_tiling_on_sc=False),
        out_type=jax.ShapeDtypeStruct((_BATCH,), jnp.float32),
        scratch_types=[
            pltpu.VMEM((_IDX_ROWS_PER_W, _IDX_MINOR), jnp.int32),
            pltpu.VMEM((_G_PAD,), jnp.float32),
            pltpu.VMEM((2, 16), jnp.float32),
            pltpu.VMEM((_B_PER_W,), jnp.float32),
            pltpu.VMEM((16, 16), jnp.float32),
            pltpu.SemaphoreType.DMA,
        ],
    )
    return f(s_flat, idx, wmat)


def kernel(X, tables, alpha):
    w = jnp.tanh(alpha).astype(jnp.float32)
    wmat = jnp.concatenate([w, jnp.zeros((6,), jnp.float32)]).reshape(2, 16)
    tbl_t = jnp.transpose(tables, (0, 2, 1))
    offs = (jnp.arange(_N_FIELDS, dtype=jnp.int32) * _VOCAB)[None, :]
    flat_idx = (X + offs).reshape(_BATCH * _N_FIELDS // _IDX_MINOR, _IDX_MINOR)
    out = _run(tbl_t, flat_idx, wmat)
    return out[:, None]
